# Initial kernel scaffold; baseline (speedup 1.0000x reference)
import functools

import jax
import jax.numpy as jnp
from jax import lax
from jax.experimental import pallas as pl
from jax.experimental.pallas import tpu as pltpu
from jax.experimental.pallas import tpu_sc as plsc

N = 10000
EG = 320000
HD = 128
H = 8
D = 16
CLAMP = 5.0

NC = 2
NS = 16
NW = NC * NS
EPW = EG // NW
C = 80
NCHUNK = EPW // C
ACCW = HD + D
RPT = N // NS


def _qkv_body(x_ref, w_ref, b_ref, q_ref, k_ref, v_ref):
    y = jnp.dot(x_ref[...], w_ref[...], preferred_element_type=jnp.float32)
    y = y + b_ref[0:1, :]
    q_ref[...] = y[:, 0:HD]
    k_ref[...] = y[:, HD:2 * HD]
    v_ref[...] = y[:, 2 * HD:3 * HD]


def _qkv(x, w_all, b_all):
    blk = 1250
    return pl.pallas_call(
        _qkv_body,
        grid=(N // blk,),
        in_specs=[
            pl.BlockSpec((blk, HD), lambda i: (i, 0)),
            pl.BlockSpec((HD, 3 * HD), lambda i: (0, 0)),
            pl.BlockSpec((8, 3 * HD), lambda i: (0, 0)),
        ],
        out_specs=[pl.BlockSpec((blk, HD), lambda i: (i, 0))] * 3,
        out_shape=[jax.ShapeDtypeStruct((N, HD), jnp.float32)] * 3,
    )(x, w_all, b_all)


def _e_body(a_ref, w_ref, b_ref, e_ref):
    y = jnp.dot(a_ref[...], w_ref[...], preferred_element_type=jnp.float32)
    e_ref[...] = y + b_ref[0:1, :]


def _eproj(edge_attr, we1, b):
    blk = 2000
    return pl.pallas_call(
        _e_body,
        grid=(EG // blk,),
        in_specs=[
            pl.BlockSpec((blk, HD), lambda i: (i, 0)),
            pl.BlockSpec((HD, HD), lambda i: (0, 0)),
            pl.BlockSpec((8, HD), lambda i: (0, 0)),
        ],
        out_specs=pl.BlockSpec((blk, HD), lambda i: (i, 0)),
        out_shape=jax.ShapeDtypeStruct((EG, HD), jnp.float32),
    )(edge_attr, we1, b)


def _edge_body(src_hbm, dst_hbm, k_hbm, q_hbm, v_hbm, e_hbm,
               we_hbm, acc_hbm,
               src_v, dst_v, kb, qb, vb, eb, web, mpb, zb, acc_sh,
               sem0, sem1, sem2, sem3):
    cid = lax.axis_index("c")
    sid = lax.axis_index("s")
    wid = sid * NC + cid

    def zrow(r, _):
        for cc in range(ACCW // 16):
            zb[r, pl.ds(cc * 16, 16)] = jnp.zeros((16,), jnp.float32)
        return 0
    lax.fori_loop(0, 125, zrow, 0)
    for m in range(5):
        pltpu.sync_copy(zb, acc_sh.at[pl.ds(sid * RPT + m * 125, 125)])
    plsc.subcore_barrier()

    pltpu.sync_copy(src_hbm.at[wid], src_v)
    pltpu.sync_copy(dst_hbm.at[wid], dst_v)

    lanes = lax.iota(jnp.int32, 16)

    def chunk(j, _):
        base = wid * EPW + j * C
        c1 = pltpu.async_copy(k_hbm.at[src_v.at[j]], kb, sem0)
        c2 = pltpu.async_copy(q_hbm.at[dst_v.at[j]], qb, sem1)
        c3 = pltpu.async_copy(v_hbm.at[src_v.at[j]], vb, sem2)
        c4 = pltpu.async_copy(e_hbm.at[pl.ds(base, C)], eb, sem3)
        c1.wait()
        c2.wait()
        c3.wait()
        c4.wait()

        def edge(i, _):
            pacc = jnp.zeros((16,), jnp.float32)
            for h in range(H):
                sl = pl.ds(h * D, D)
                s = kb[i, sl] * qb[i, sl] * eb[i, sl]
                web[i, sl] = s
                t = jnp.clip(jnp.sum(s) * 0.25, -CLAMP, CLAMP)
                pv = jnp.exp(jnp.broadcast_to(t, (16,)))
                mpb[i, sl] = (vb[i, sl] + s) * pv
                pacc = jnp.where(lanes == h, pv, pacc)
            mpb[i, pl.ds(HD, 16)] = pacc
            return 0
        lax.fori_loop(0, C, edge, 0)

        pltpu.sync_copy(web, we_hbm.at[pl.ds(base, C)])
        pltpu.sync_copy(mpb, acc_sh.at[dst_v.at[j]], add=True)
        return 0
    lax.fori_loop(0, NCHUNK, chunk, 0)

    plsc.subcore_barrier()
    pltpu.sync_copy(acc_sh.at[pl.ds(sid * RPT, RPT)],
                    acc_hbm.at[cid, pl.ds(sid * RPT, RPT)])


def _edges(src_r, dst_r, k, q, v, e):
    mesh = plsc.VectorSubcoreMesh(core_axis_name="c", subcore_axis_name="s")
    f = pl.kernel(
        _edge_body,
        out_type=[
            jax.ShapeDtypeStruct((EG, HD), jnp.float32),
            jax.ShapeDtypeStruct((NC, N, ACCW), jnp.float32),
        ],
        mesh=mesh,
        scratch_types=[
            pltpu.VMEM((NCHUNK, C), jnp.int32),
            pltpu.VMEM((NCHUNK, C), jnp.int32),
            pltpu.VMEM((C, HD), jnp.float32),
            pltpu.VMEM((C, HD), jnp.float32),
            pltpu.VMEM((C, HD), jnp.float32),
            pltpu.VMEM((C, HD), jnp.float32),
            pltpu.VMEM((C, HD), jnp.float32),
            pltpu.VMEM((C, ACCW), jnp.float32),
            pltpu.VMEM((125, ACCW), jnp.float32),
            pltpu.VMEM_SHARED((N, ACCW), jnp.float32),
            pltpu.SemaphoreType.DMA,
            pltpu.SemaphoreType.DMA,
            pltpu.SemaphoreType.DMA,
            pltpu.SemaphoreType.DMA,
        ],
    )
    return f(src_r, dst_r, k, q, v, e)


def _comb_body(a_ref, wv_ref):
    a = a_ref[0] + a_ref[1]
    num = a[:, 0:HD]
    den = a[:, HD:HD + H] + 1e-16
    rep = (lax.broadcasted_iota(jnp.int32, (H, HD), 1) // D ==
           lax.broadcasted_iota(jnp.int32, (H, HD), 0)).astype(jnp.float32)
    wv_ref[...] = num / jnp.dot(den, rep, preferred_element_type=jnp.float32)


def _combine(acc):
    blk = 1250
    return pl.pallas_call(
        _comb_body,
        grid=(N // blk,),
        in_specs=[pl.BlockSpec((NC, blk, ACCW), lambda i: (0, i, 0))],
        out_specs=pl.BlockSpec((blk, HD), lambda i: (i, 0)),
        out_shape=jax.ShapeDtypeStruct((N, HD), jnp.float32),
    )(acc)


def kernel(x, edge_index, edge_attr, WQ, bQ, WK, WE1, bE1, WV):
    w_all = jnp.concatenate([WQ, WK, WV], axis=1)
    zb = jnp.zeros_like(bQ)
    b_all = jnp.tile(jnp.concatenate([bQ, zb, zb])[None, :], (8, 1))
    q, k, v = _qkv(x, w_all, b_all)
    e = _eproj(edge_attr, WE1, jnp.tile(bE1[None, :], (8, 1)))
    src_r = edge_index[0].reshape(NW, NCHUNK, C)
    dst_r = edge_index[1].reshape(NW, NCHUNK, C)
    we, acc = _edges(src_r, dst_r, k, q, v, e)
    wv = _combine(acc)
    return wv.reshape(N, H, D), we


# speedup vs baseline: 72.8015x; 72.8015x over previous
import functools

import jax
import jax.numpy as jnp
from jax import lax
from jax.experimental import pallas as pl
from jax.experimental.pallas import tpu as pltpu
from jax.experimental.pallas import tpu_sc as plsc

N = 10000
EG = 320000
HD = 128
H = 8
D = 16
CLAMP = 5.0

NC = 2
NS = 16
NW = NC * NS
EPW = EG // NW
C = 40
NCHUNK = EPW // C
NACC = 10240
RPT = NACC // NS
NP = NACC * H
NPR = NP // HD


def _qkv_body(x_ref, w_ref, b_ref, q_ref, k_ref, v_ref):
    y = jnp.dot(x_ref[...], w_ref[...], preferred_element_type=jnp.float32)
    y = y + b_ref[0:1, :]
    q_ref[...] = y[:, 0:HD]
    k_ref[...] = y[:, HD:2 * HD]
    v_ref[...] = y[:, 2 * HD:3 * HD]


def _qkv(x, w_all, b_all):
    blk = 1000
    return pl.pallas_call(
        _qkv_body,
        grid=(N // blk,),
        in_specs=[
            pl.BlockSpec((blk, HD), lambda i: (i, 0)),
            pl.BlockSpec((HD, 3 * HD), lambda i: (0, 0)),
            pl.BlockSpec((8, 3 * HD), lambda i: (0, 0)),
        ],
        out_specs=[pl.BlockSpec((blk, HD), lambda i: (i, 0))] * 3,
        out_shape=[jax.ShapeDtypeStruct((N, HD), jnp.float32)] * 3,
    )(x, w_all, b_all)


def _e_body(a_ref, w_ref, b_ref, e_ref):
    y = jnp.dot(a_ref[...], w_ref[...], preferred_element_type=jnp.float32)
    e_ref[...] = y + b_ref[0:1, :]


def _eproj(edge_attr, we1, b):
    blk = 2000
    return pl.pallas_call(
        _e_body,
        grid=(EG // blk,),
        in_specs=[
            pl.BlockSpec((blk, HD), lambda i: (i, 0)),
            pl.BlockSpec((HD, HD), lambda i: (0, 0)),
            pl.BlockSpec((8, HD), lambda i: (0, 0)),
        ],
        out_specs=pl.BlockSpec((blk, HD), lambda i: (i, 0)),
        out_shape=jax.ShapeDtypeStruct((EG, HD), jnp.float32),
    )(edge_attr, we1, b)


def _edge_body(src_hbm, dst_hbm, k_hbm, q_hbm, v_hbm, e_hbm,
               we_hbm, acc_hbm, pacc_hbm,
               acc_sh, acc_p, sem0, sem1, sem2, sem3):
    def scoped(src_c, dst_c, dst_cp, d16, kb, qb, vb, eb, web, mb, pb):
        _edge_inner(src_hbm, dst_hbm, k_hbm, q_hbm, v_hbm, e_hbm,
                    we_hbm, acc_hbm, pacc_hbm,
                    src_c, dst_c, dst_cp, d16, kb, qb, vb, eb, web, mb, pb,
                    acc_sh, acc_p, sem0, sem1, sem2, sem3)
    pl.run_scoped(
        scoped,
        pltpu.VMEM((C,), jnp.int32),
        pltpu.VMEM((C,), jnp.int32),
        pltpu.VMEM((C + 16,), jnp.int32),
        pltpu.VMEM((C,), jnp.int32),
        pltpu.VMEM((C, HD), jnp.float32),
        pltpu.VMEM((C, HD), jnp.float32),
        pltpu.VMEM((C, HD), jnp.float32),
        pltpu.VMEM((C, HD), jnp.float32),
        pltpu.VMEM((C, HD), jnp.float32),
        pltpu.VMEM((C, HD), jnp.float32),
        pltpu.VMEM((C, HD), jnp.float32),
    )


def _edge_inner(src_hbm, dst_hbm, k_hbm, q_hbm, v_hbm, e_hbm,
                we_hbm, acc_hbm, pacc_hbm,
                src_c, dst_c, dst_cp, d16, kb, qb, vb, eb, web, mb, pb,
                acc_sh, acc_p, sem0, sem1, sem2, sem3):
    cid = lax.axis_index("c")
    sid = lax.axis_index("s")
    wid = sid * NC + cid

    lanes = lax.iota(jnp.int32, 16)
    z16 = jnp.zeros((16,), jnp.float32)

    def zrow(r, _):
        for cc in range(HD // 16):
            mb[r, pl.ds(cc * 16, 16)] = z16
        return 0
    lax.fori_loop(0, C, zrow, 0)
    for m in range(RPT // C):
        pltpu.sync_copy(mb, acc_sh.at[pl.ds(sid * RPT + m * C, C)])
    pltpu.sync_copy(mb, acc_p.at[pl.ds(sid * (NPR // NS), NPR // NS)])
    plsc.subcore_barrier()

    def chunk(j, _):
        base = wid * EPW + j * C
        ci = pltpu.async_copy(src_hbm.at[pl.ds(base, C)], src_c, sem3)
        cj = pltpu.async_copy(dst_hbm.at[pl.ds(base, C)], dst_c, sem3)
        ck = pltpu.async_copy(dst_hbm.at[pl.ds(base, C)],
                              dst_cp.at[pl.ds(0, C)], sem3)
        ci.wait()
        cj.wait()
        ck.wait()
        c1 = pltpu.async_copy(k_hbm.at[src_c], kb, sem0)
        c2 = pltpu.async_copy(q_hbm.at[dst_c], qb, sem1)
        c3 = pltpu.async_copy(v_hbm.at[src_c], vb, sem2)
        c4 = pltpu.async_copy(e_hbm.at[pl.ds(base, C)], eb, sem3)
        c1.wait()
        c2.wait()
        c3.wait()
        c4.wait()

        for g in range((C + 15) // 16):
            idxs = lanes + g * 16
            dvec = dst_cp[pl.ds(g * 16, 16)]
            plsc.store_scatter(d16, [idxs], dvec >> 4, mask=idxs < C)

        def edge(i, _):
            dstn = dst_cp[pl.ds(i, 16)][0]
            slot = (dstn & 15) >> 1
            odd = dstn & 1
            pv_lo = z16
            pv_hi = z16
            for h in range(H):
                sl = pl.ds(h * D, D)
                s = kb[i, sl] * qb[i, sl] * eb[i, sl]
                web[i, sl] = s
                t = jnp.clip(plsc.cumsum(s)[15] * 0.25, -CLAMP, CLAMP)
                pv = jnp.exp(jnp.broadcast_to(t, (16,)))
                mb[i, sl] = (vb[i, sl] + s) * pv
                pv_lo = jnp.where(lanes == h, pv, pv_lo)
                pv_hi = jnp.where(lanes == h + H, pv, pv_hi)
            for cc in range(HD // 16):
                pb[i, pl.ds(cc * 16, 16)] = z16
            pb[i, pl.ds(slot * 16, 16)] = jnp.where(odd == 1, pv_hi, pv_lo)
            return 0
        lax.fori_loop(0, C, edge, 0)

        pltpu.sync_copy(web, we_hbm.at[pl.ds(base, C)])
        pltpu.sync_copy(mb, acc_sh.at[dst_c], add=True)
        pltpu.sync_copy(pb, acc_p.at[d16], add=True)
        return 0
    lax.fori_loop(0, NCHUNK, chunk, 0)

    plsc.subcore_barrier()
    pltpu.sync_copy(acc_sh.at[pl.ds(sid * RPT, RPT)],
                    acc_hbm.at[pl.ds(cid * NACC + sid * RPT, RPT)])
    pltpu.sync_copy(acc_p.at[pl.ds(sid * (NPR // NS), NPR // NS)],
                    pacc_hbm.at[pl.ds(cid * NPR + sid * (NPR // NS),
                                      NPR // NS)])


def _edges(src_r, dst_r, k, q, v, e):
    mesh = plsc.VectorSubcoreMesh(core_axis_name="c", subcore_axis_name="s")
    f = pl.kernel(
        _edge_body,
        out_type=[
            jax.ShapeDtypeStruct((EG, HD), jnp.float32),
            jax.ShapeDtypeStruct((NC * NACC, HD), jnp.float32),
            jax.ShapeDtypeStruct((NC * NPR, HD), jnp.float32),
        ],
        mesh=mesh,
        compiler_params=pltpu.CompilerParams(needs_layout_passes=False),
        scratch_types=[
            pltpu.VMEM_SHARED((NACC, HD), jnp.float32),
            pltpu.VMEM_SHARED((NPR, HD), jnp.float32),
            pltpu.SemaphoreType.DMA,
            pltpu.SemaphoreType.DMA,
            pltpu.SemaphoreType.DMA,
            pltpu.SemaphoreType.DMA,
        ],
    )
    return f(src_r, dst_r, k, q, v, e)


def _comb_body(a_ref, p_ref, wv_ref):
    num = a_ref[0] + a_ref[1]
    den = jnp.sum(p_ref[...], axis=0) + 1e-16
    rep = (lax.broadcasted_iota(jnp.int32, (H, HD), 1) // D ==
           lax.broadcasted_iota(jnp.int32, (H, HD), 0)).astype(jnp.float32)
    wv_ref[...] = num / jnp.dot(den, rep, preferred_element_type=jnp.float32)


def _combine(acc, paccs):
    blk = 1000
    return pl.pallas_call(
        _comb_body,
        grid=(N // blk,),
        in_specs=[
            pl.BlockSpec((NC, blk, HD), lambda i: (0, i, 0)),
            pl.BlockSpec((NC, blk, H), lambda i: (0, i, 0)),
        ],
        out_specs=pl.BlockSpec((blk, HD), lambda i: (i, 0)),
        out_shape=jax.ShapeDtypeStruct((N, HD), jnp.float32),
    )(acc, paccs)


def kernel(x, edge_index, edge_attr, WQ, bQ, WK, WE1, bE1, WV):
    w_all = jnp.concatenate([WQ, WK, WV], axis=1)
    zb = jnp.zeros_like(bQ)
    b_all = jnp.tile(jnp.concatenate([bQ, zb, zb])[None, :], (8, 1))
    q, k, v = _qkv(x, w_all, b_all)
    e = _eproj(edge_attr, WE1, jnp.tile(bE1[None, :], (8, 1)))
    src_r = edge_index[0]
    dst_r = edge_index[1]
    we, acc, paccs = _edges(src_r, dst_r, k, q, v, e)
    wv = _combine(acc.reshape(NC, NACC, HD), paccs.reshape(NC, NACC, H))
    return wv.reshape(N, H, D), we
